# baseline (speedup 1.0000x reference)
import functools

import jax
import jax.numpy as jnp
from jax import lax
from jax.experimental import pallas as pl
from jax.experimental.pallas import tpu as pltpu
from jax.experimental.pallas import tpu_sc as plsc

_B = 16
_N = 20000
_IOU_THRESH = 0.2
_NC = 2
_NS = 16
_NW = _NC * _NS
_SC_NT = 64
_TC_START = _SC_NT * 128
_TC_BLK = 4096
_TC_GRID = (_N - _TC_START + _TC_BLK - 1) // _TC_BLK
_FIELDS = (0, 1, 2, 3, 6)


def _sc_partial_max(boxes_t, gt8):
    mesh = plsc.VectorSubcoreMesh(core_axis_name="c", subcore_axis_name="s")

    @functools.partial(
        pl.kernel,
        mesh=mesh,
        out_type=jax.ShapeDtypeStruct((_NW, 2, 128), jnp.float32),
        compiler_params=pltpu.CompilerParams(
            needs_layout_passes=False, use_tc_tiling_on_sc=True),
        scratch_types=[
            pltpu.VMEM((2, 4, _B, 128), jnp.float32),
            pltpu.VMEM((2, _B, 128), jnp.float32),
            pltpu.VMEM((_B, 128), jnp.float32),
            pltpu.VMEM((_B, 16), jnp.float32),
            pltpu.VMEM((2, 128), jnp.float32),
            pltpu.SemaphoreType.DMA,
            pltpu.SemaphoreType.DMA,
        ],
    )
    def k(boxes_hbm, gt_hbm, out_hbm, bv, cv, gt_v, acc_v,
          flat_v, sem0, sem1):
        cid = lax.axis_index("c")
        sid = lax.axis_index("s")
        wid = sid * _NC + cid

        pltpu.sync_copy(gt_hbm, gt_v)

        sems = (sem0, sem1)

        def start_tile(t, slot):
            pltpu.async_copy(
                boxes_hbm.at[pl.ds(0, 4), :, pl.ds(t * 128, 128)],
                bv.at[slot], sems[slot])
            pltpu.async_copy(
                boxes_hbm.at[6, :, pl.ds(t * 128, 128)],
                cv.at[slot], sems[slot])

        def wait_tile(t, slot):
            pltpu.make_async_copy(
                boxes_hbm.at[pl.ds(0, 4), :, pl.ds(t * 128, 128)],
                bv.at[slot], sems[slot]).wait()
            pltpu.make_async_copy(
                boxes_hbm.at[6, :, pl.ds(t * 128, 128)],
                cv.at[slot], sems[slot]).wait()

        neg = jnp.full((16,), -1e9, jnp.float32)
        c0 = jnp.full((16,), 0, jnp.int32)
        c1 = jnp.full((16,), 1, jnp.int32)
        c2 = jnp.full((16,), 2, jnp.int32)
        c3 = jnp.full((16,), 3, jnp.int32)

        def init_acc(b, _):
            acc_v[b] = neg
            return 0

        lax.fori_loop(0, _B, init_acc, 0)

        start_tile(wid, 0)
        start_tile(wid + _NW, 1)

        def process_tile(t, slot):
            wait_tile(t, slot)

            def img_body(b, _):
                brow = jnp.full((16,), b, jnp.int32)
                gx1 = plsc.load_gather(gt_v, [brow, c0])
                gy1 = plsc.load_gather(gt_v, [brow, c1])
                gx2 = plsc.load_gather(gt_v, [brow, c2])
                gy2 = plsc.load_gather(gt_v, [brow, c3])
                agv = (gx2 - gx1) * (gy2 - gy1)

                acc = acc_v[b]
                for c in range(8):
                    sl = pl.ds(c * 16, 16)
                    x = bv[slot, 0, b, sl]
                    y = bv[slot, 1, b, sl]
                    w = bv[slot, 2, b, sl]
                    h = bv[slot, 3, b, sl]
                    cl = cv[slot, b, sl]
                    wh = w * 0.5
                    hh = h * 0.5
                    bx1 = x - wh
                    bx2 = x + wh
                    by1 = y - hh
                    by2 = y + hh
                    ix1 = jnp.maximum(bx1, gx1)
                    iy1 = jnp.maximum(by1, gy1)
                    ix2 = jnp.minimum(bx2, gx2)
                    iy2 = jnp.minimum(by2, gy2)
                    dx = jnp.maximum(ix2 - ix1, 0.0)
                    dy = jnp.maximum(iy2 - iy1, 0.0)
                    inter = dx * dy
                    area_b = w * h
                    iou = inter / ((area_b + agv) - inter)
                    val = jnp.where(cl == 0.0, iou, neg)
                    acc = jnp.maximum(acc, val)
                acc_v[b] = acc
                return 0

            lax.fori_loop(0, _B, img_body, 0)

        process_tile(wid, 0)
        process_tile(wid + _NW, 1)

        def out_body(b, _):
            flat_v[b // 8, pl.ds((b % 8) * 16, 16)] = acc_v[b]
            return 0

        lax.fori_loop(0, _B, out_body, 0)
        pltpu.sync_copy(flat_v, out_hbm.at[wid])

    return k(boxes_t, gt8)


def _tc_range_body(x_ref, y_ref, w_ref, h_ref, c_ref, g_ref, tcp_ref,
                   acc_ref):
    g = pl.program_id(0)

    @pl.when(g == 0)
    def _():
        acc_ref[...] = jnp.full((_B, _TC_BLK), -1e9, jnp.float32)

    gx1 = g_ref[:, 0:1]
    gy1 = g_ref[:, 1:2]
    gx2 = g_ref[:, 2:3]
    gy2 = g_ref[:, 3:4]
    x = x_ref[0]
    y = y_ref[0]
    w = w_ref[0]
    h = h_ref[0]
    cl = c_ref[0]
    wh = w * 0.5
    hh = h * 0.5
    ix1 = jnp.maximum(x - wh, gx1)
    iy1 = jnp.maximum(y - hh, gy1)
    ix2 = jnp.minimum(x + wh, gx2)
    iy2 = jnp.minimum(y + hh, gy2)
    inter = jnp.maximum(ix2 - ix1, 0.0) * jnp.maximum(iy2 - iy1, 0.0)
    area_g = (gx2 - gx1) * (gy2 - gy1)
    iou = inter / ((w * h + area_g) - inter)
    n = (_TC_START + g * _TC_BLK
         + jax.lax.broadcasted_iota(jnp.int32, (_B, _TC_BLK), 1))
    ok = (cl == 0.0) & (n < _N)
    acc_ref[...] = jnp.maximum(acc_ref[...], jnp.where(ok, iou, -1e9))

    @pl.when(g == _TC_GRID - 1)
    def _():
        tcp_ref[...] = jnp.max(acc_ref[...], axis=1).reshape(1, _B)


def _tc_finish_body(p_ref, tcp_ref, mp_ref, dl_ref):
    p = jnp.max(p_ref[...], axis=0)
    pmax = jnp.max(p.reshape(2, 8, 16), axis=2).reshape(16)
    m = jnp.maximum(pmax, tcp_ref[0, :])
    m = jnp.where(m >= _IOU_THRESH, m, 0.0)
    mp_ref[...] = m.reshape(1, _B)
    dl_ref[...] = jnp.sum(m).reshape(1, 1) * (1.0 / _B)


def kernel(boxes, gt):
    boxes_t = jnp.transpose(boxes, (2, 0, 1))
    gt_n = gt * (1.0 / 416.0)
    gt8 = jnp.pad(gt_n, ((0, 0), (0, 124)))

    partials = _sc_partial_max(boxes_t, gt8)

    def field_spec(f):
        return pl.BlockSpec((1, _B, _TC_BLK),
                            lambda g, f=f: (f, 0, g + _TC_START // _TC_BLK))

    tcp = pl.pallas_call(
        _tc_range_body,
        grid=(_TC_GRID,),
        in_specs=[field_spec(f) for f in _FIELDS]
        + [pl.BlockSpec((_B, 128), lambda g: (0, 0))],
        out_specs=pl.BlockSpec((1, _B), lambda g: (0, 0)),
        out_shape=jax.ShapeDtypeStruct((1, _B), jnp.float32),
        scratch_shapes=[pltpu.VMEM((_B, _TC_BLK), jnp.float32)],
    )(boxes_t, boxes_t, boxes_t, boxes_t, boxes_t, gt8)

    mp, dl = pl.pallas_call(
        _tc_finish_body,
        out_shape=[
            jax.ShapeDtypeStruct((1, _B), jnp.float32),
            jax.ShapeDtypeStruct((1, 1), jnp.float32),
        ],
    )(partials, tcp)
    return dl[0, 0], mp.reshape(_B)


# speedup vs baseline: 1.0804x; 1.0804x over previous
import functools

import jax
import jax.numpy as jnp
from jax import lax
from jax.experimental import pallas as pl
from jax.experimental.pallas import tpu as pltpu
from jax.experimental.pallas import tpu_sc as plsc

_B = 16
_N = 20000
_IOU_THRESH = 0.2
_NC = 2
_NS = 16
_NW = _NC * _NS
_SC_NT = 32
_TC_START = _SC_NT * 128
_TC_BLK = 4096
_TC_GRID = (_N - _TC_START + _TC_BLK - 1) // _TC_BLK
_FIELDS = (0, 1, 2, 3, 6)


def _sc_partial_max(boxes_t, gt8):
    mesh = plsc.VectorSubcoreMesh(core_axis_name="c", subcore_axis_name="s")

    @functools.partial(
        pl.kernel,
        mesh=mesh,
        out_type=jax.ShapeDtypeStruct((_NW, 2, 128), jnp.float32),
        compiler_params=pltpu.CompilerParams(
            needs_layout_passes=False, use_tc_tiling_on_sc=True),
        scratch_types=[
            pltpu.VMEM((2, 4, _B, 128), jnp.float32),
            pltpu.VMEM((2, _B, 128), jnp.float32),
            pltpu.VMEM((_B, 128), jnp.float32),
            pltpu.VMEM((_B, 16), jnp.float32),
            pltpu.VMEM((2, 128), jnp.float32),
            pltpu.SemaphoreType.DMA,
            pltpu.SemaphoreType.DMA,
        ],
    )
    def k(boxes_hbm, gt_hbm, out_hbm, bv, cv, gt_v, acc_v,
          flat_v, sem0, sem1):
        cid = lax.axis_index("c")
        sid = lax.axis_index("s")
        wid = sid * _NC + cid

        pltpu.sync_copy(gt_hbm, gt_v)

        sems = (sem0, sem1)

        def start_tile(t, slot):
            pltpu.async_copy(
                boxes_hbm.at[pl.ds(0, 4), :, pl.ds(t * 128, 128)],
                bv.at[slot], sems[slot])
            pltpu.async_copy(
                boxes_hbm.at[6, :, pl.ds(t * 128, 128)],
                cv.at[slot], sems[slot])

        def wait_tile(t, slot):
            pltpu.make_async_copy(
                boxes_hbm.at[pl.ds(0, 4), :, pl.ds(t * 128, 128)],
                bv.at[slot], sems[slot]).wait()
            pltpu.make_async_copy(
                boxes_hbm.at[6, :, pl.ds(t * 128, 128)],
                cv.at[slot], sems[slot]).wait()

        neg = jnp.full((16,), -1e9, jnp.float32)
        c0 = jnp.full((16,), 0, jnp.int32)
        c1 = jnp.full((16,), 1, jnp.int32)
        c2 = jnp.full((16,), 2, jnp.int32)
        c3 = jnp.full((16,), 3, jnp.int32)

        def init_acc(b, _):
            acc_v[b] = neg
            return 0

        lax.fori_loop(0, _B, init_acc, 0)

        start_tile(wid, 0)

        def process_tile(t, slot):
            wait_tile(t, slot)

            def img_body(b, _):
                brow = jnp.full((16,), b, jnp.int32)
                gx1 = plsc.load_gather(gt_v, [brow, c0])
                gy1 = plsc.load_gather(gt_v, [brow, c1])
                gx2 = plsc.load_gather(gt_v, [brow, c2])
                gy2 = plsc.load_gather(gt_v, [brow, c3])
                agv = (gx2 - gx1) * (gy2 - gy1)

                acc = acc_v[b]
                for c in range(8):
                    sl = pl.ds(c * 16, 16)
                    x = bv[slot, 0, b, sl]
                    y = bv[slot, 1, b, sl]
                    w = bv[slot, 2, b, sl]
                    h = bv[slot, 3, b, sl]
                    cl = cv[slot, b, sl]
                    wh = w * 0.5
                    hh = h * 0.5
                    bx1 = x - wh
                    bx2 = x + wh
                    by1 = y - hh
                    by2 = y + hh
                    ix1 = jnp.maximum(bx1, gx1)
                    iy1 = jnp.maximum(by1, gy1)
                    ix2 = jnp.minimum(bx2, gx2)
                    iy2 = jnp.minimum(by2, gy2)
                    dx = jnp.maximum(ix2 - ix1, 0.0)
                    dy = jnp.maximum(iy2 - iy1, 0.0)
                    inter = dx * dy
                    area_b = w * h
                    iou = inter / ((area_b + agv) - inter)
                    val = jnp.where(cl == 0.0, iou, neg)
                    acc = jnp.maximum(acc, val)
                acc_v[b] = acc
                return 0

            lax.fori_loop(0, _B, img_body, 0)

        process_tile(wid, 0)

        def out_body(b, _):
            flat_v[b // 8, pl.ds((b % 8) * 16, 16)] = acc_v[b]
            return 0

        lax.fori_loop(0, _B, out_body, 0)
        pltpu.sync_copy(flat_v, out_hbm.at[wid])

    return k(boxes_t, gt8)


def _tc_range_body(x_ref, y_ref, w_ref, h_ref, c_ref, g_ref, tcp_ref,
                   acc_ref):
    g = pl.program_id(0)

    @pl.when(g == 0)
    def _():
        acc_ref[...] = jnp.full((_B, _TC_BLK), -1e9, jnp.float32)

    gx1 = g_ref[:, 0:1]
    gy1 = g_ref[:, 1:2]
    gx2 = g_ref[:, 2:3]
    gy2 = g_ref[:, 3:4]
    x = x_ref[0]
    y = y_ref[0]
    w = w_ref[0]
    h = h_ref[0]
    cl = c_ref[0]
    wh = w * 0.5
    hh = h * 0.5
    ix1 = jnp.maximum(x - wh, gx1)
    iy1 = jnp.maximum(y - hh, gy1)
    ix2 = jnp.minimum(x + wh, gx2)
    iy2 = jnp.minimum(y + hh, gy2)
    inter = jnp.maximum(ix2 - ix1, 0.0) * jnp.maximum(iy2 - iy1, 0.0)
    area_g = (gx2 - gx1) * (gy2 - gy1)
    iou = inter / ((w * h + area_g) - inter)
    n = (_TC_START + g * _TC_BLK
         + jax.lax.broadcasted_iota(jnp.int32, (_B, _TC_BLK), 1))
    ok = (cl == 0.0) & (n < _N)
    acc_ref[...] = jnp.maximum(acc_ref[...], jnp.where(ok, iou, -1e9))

    @pl.when(g == _TC_GRID - 1)
    def _():
        tcp_ref[...] = jnp.max(acc_ref[...], axis=1).reshape(1, _B)


def _tc_finish_body(p_ref, tcp_ref, mp_ref, dl_ref):
    p = jnp.max(p_ref[...], axis=0)
    pmax = jnp.max(p.reshape(2, 8, 16), axis=2).reshape(16)
    m = jnp.maximum(pmax, tcp_ref[0, :])
    m = jnp.where(m >= _IOU_THRESH, m, 0.0)
    mp_ref[...] = m.reshape(1, _B)
    dl_ref[...] = jnp.sum(m).reshape(1, 1) * (1.0 / _B)


def kernel(boxes, gt):
    boxes_t = jnp.transpose(boxes, (2, 0, 1))
    gt_n = gt * (1.0 / 416.0)
    gt8 = jnp.pad(gt_n, ((0, 0), (0, 124)))

    partials = _sc_partial_max(boxes_t, gt8)

    def field_spec(f):
        return pl.BlockSpec((1, _B, _TC_BLK),
                            lambda g, f=f: (f, 0, g + _TC_START // _TC_BLK))

    tcp = pl.pallas_call(
        _tc_range_body,
        grid=(_TC_GRID,),
        in_specs=[field_spec(f) for f in _FIELDS]
        + [pl.BlockSpec((_B, 128), lambda g: (0, 0))],
        out_specs=pl.BlockSpec((1, _B), lambda g: (0, 0)),
        out_shape=jax.ShapeDtypeStruct((1, _B), jnp.float32),
        scratch_shapes=[pltpu.VMEM((_B, _TC_BLK), jnp.float32)],
    )(boxes_t, boxes_t, boxes_t, boxes_t, boxes_t, gt8)

    mp, dl = pl.pallas_call(
        _tc_finish_body,
        out_shape=[
            jax.ShapeDtypeStruct((1, _B), jnp.float32),
            jax.ShapeDtypeStruct((1, 1), jnp.float32),
        ],
    )(partials, tcp)
    return dl[0, 0], mp.reshape(_B)


# speedup vs baseline: 1.0806x; 1.0002x over previous
import functools

import jax
import jax.numpy as jnp
from jax import lax
from jax.experimental import pallas as pl
from jax.experimental.pallas import tpu as pltpu
from jax.experimental.pallas import tpu_sc as plsc

_B = 16
_N = 20000
_IOU_THRESH = 0.2
_NC = 2
_NS = 16
_NW = _NC * _NS
_SC_NT = 32
_TC_START = _SC_NT * 128
_TC_BLK = 4096
_TC_GRID = (_N - _TC_START + _TC_BLK - 1) // _TC_BLK
_FIELDS = (0, 1, 2, 3, 6)


def _sc_partial_max(boxes_t, gt8):
    mesh = plsc.VectorSubcoreMesh(core_axis_name="c", subcore_axis_name="s")

    @functools.partial(
        pl.kernel,
        mesh=mesh,
        out_type=jax.ShapeDtypeStruct((_NW, 2, 128), jnp.float32),
        compiler_params=pltpu.CompilerParams(
            needs_layout_passes=False, use_tc_tiling_on_sc=True),
        scratch_types=[
            pltpu.VMEM((2, 4, _B, 128), jnp.float32),
            pltpu.VMEM((2, _B, 128), jnp.float32),
            pltpu.VMEM((_B, 128), jnp.float32),
            pltpu.VMEM((2, 128), jnp.float32),
            pltpu.SemaphoreType.DMA,
            pltpu.SemaphoreType.DMA,
        ],
    )
    def k(boxes_hbm, gt_hbm, out_hbm, bv, cv, gt_v, flat_v, sem0, sem1):
        cid = lax.axis_index("c")
        sid = lax.axis_index("s")
        wid = sid * _NC + cid

        pltpu.sync_copy(gt_hbm, gt_v)

        sems = (sem0, sem1)

        def start_tile(t, slot):
            pltpu.async_copy(
                boxes_hbm.at[pl.ds(0, 4), :, pl.ds(t * 128, 128)],
                bv.at[slot], sems[slot])
            pltpu.async_copy(
                boxes_hbm.at[6, :, pl.ds(t * 128, 128)],
                cv.at[slot], sems[slot])

        def wait_tile(t, slot):
            pltpu.make_async_copy(
                boxes_hbm.at[pl.ds(0, 4), :, pl.ds(t * 128, 128)],
                bv.at[slot], sems[slot]).wait()
            pltpu.make_async_copy(
                boxes_hbm.at[6, :, pl.ds(t * 128, 128)],
                cv.at[slot], sems[slot]).wait()

        neg = jnp.full((16,), -1e9, jnp.float32)
        c0 = jnp.full((16,), 0, jnp.int32)
        c1 = jnp.full((16,), 1, jnp.int32)
        c2 = jnp.full((16,), 2, jnp.int32)
        c3 = jnp.full((16,), 3, jnp.int32)

        start_tile(wid, 0)

        def process_tile(t, slot):
            wait_tile(t, slot)

            def img_body(b, _):
                brow = jnp.full((16,), b, jnp.int32)
                gx1 = plsc.load_gather(gt_v, [brow, c0])
                gy1 = plsc.load_gather(gt_v, [brow, c1])
                gx2 = plsc.load_gather(gt_v, [brow, c2])
                gy2 = plsc.load_gather(gt_v, [brow, c3])
                agv = (gx2 - gx1) * (gy2 - gy1)

                acc = neg
                for c in range(8):
                    sl = pl.ds(c * 16, 16)
                    x = bv[slot, 0, b, sl]
                    y = bv[slot, 1, b, sl]
                    w = bv[slot, 2, b, sl]
                    h = bv[slot, 3, b, sl]
                    cl = cv[slot, b, sl]
                    wh = w * 0.5
                    hh = h * 0.5
                    bx1 = x - wh
                    bx2 = x + wh
                    by1 = y - hh
                    by2 = y + hh
                    ix1 = jnp.maximum(bx1, gx1)
                    iy1 = jnp.maximum(by1, gy1)
                    ix2 = jnp.minimum(bx2, gx2)
                    iy2 = jnp.minimum(by2, gy2)
                    dx = jnp.maximum(ix2 - ix1, 0.0)
                    dy = jnp.maximum(iy2 - iy1, 0.0)
                    inter = dx * dy
                    area_b = w * h
                    iou = inter / ((area_b + agv) - inter)
                    val = jnp.where(cl == 0.0, iou, neg)
                    acc = jnp.maximum(acc, val)
                flat_v[b // 8, pl.ds((b % 8) * 16, 16)] = acc
                return 0

            lax.fori_loop(0, _B, img_body, 0)

        process_tile(wid, 0)
        pltpu.sync_copy(flat_v, out_hbm.at[wid])

    return k(boxes_t, gt8)


def _tc_range_body(x_ref, y_ref, w_ref, h_ref, c_ref, g_ref, tcp_ref,
                   acc_ref):
    g = pl.program_id(0)

    @pl.when(g == 0)
    def _():
        acc_ref[...] = jnp.full((_B, _TC_BLK), -1e9, jnp.float32)

    gx1 = g_ref[:, 0:1]
    gy1 = g_ref[:, 1:2]
    gx2 = g_ref[:, 2:3]
    gy2 = g_ref[:, 3:4]
    x = x_ref[0]
    y = y_ref[0]
    w = w_ref[0]
    h = h_ref[0]
    cl = c_ref[0]
    wh = w * 0.5
    hh = h * 0.5
    ix1 = jnp.maximum(x - wh, gx1)
    iy1 = jnp.maximum(y - hh, gy1)
    ix2 = jnp.minimum(x + wh, gx2)
    iy2 = jnp.minimum(y + hh, gy2)
    inter = jnp.maximum(ix2 - ix1, 0.0) * jnp.maximum(iy2 - iy1, 0.0)
    area_g = (gx2 - gx1) * (gy2 - gy1)
    iou = inter / ((w * h + area_g) - inter)
    n = (_TC_START + g * _TC_BLK
         + jax.lax.broadcasted_iota(jnp.int32, (_B, _TC_BLK), 1))
    ok = (cl == 0.0) & (n < _N)
    acc_ref[...] = jnp.maximum(acc_ref[...], jnp.where(ok, iou, -1e9))

    @pl.when(g == _TC_GRID - 1)
    def _():
        tcp_ref[...] = jnp.max(acc_ref[...], axis=1).reshape(1, _B)


def _tc_finish_body(p_ref, tcp_ref, mp_ref, dl_ref):
    p = jnp.max(p_ref[...], axis=0)
    pmax = jnp.max(p.reshape(2, 8, 16), axis=2).reshape(16)
    m = jnp.maximum(pmax, tcp_ref[0, :])
    m = jnp.where(m >= _IOU_THRESH, m, 0.0)
    mp_ref[...] = m.reshape(1, _B)
    dl_ref[...] = jnp.sum(m).reshape(1, 1) * (1.0 / _B)


def kernel(boxes, gt):
    boxes_t = jnp.transpose(boxes, (2, 0, 1))
    gt_n = gt * (1.0 / 416.0)
    gt8 = jnp.pad(gt_n, ((0, 0), (0, 124)))

    partials = _sc_partial_max(boxes_t, gt8)

    def field_spec(f):
        return pl.BlockSpec((1, _B, _TC_BLK),
                            lambda g, f=f: (f, 0, g + _TC_START // _TC_BLK))

    tcp = pl.pallas_call(
        _tc_range_body,
        grid=(_TC_GRID,),
        in_specs=[field_spec(f) for f in _FIELDS]
        + [pl.BlockSpec((_B, 128), lambda g: (0, 0))],
        out_specs=pl.BlockSpec((1, _B), lambda g: (0, 0)),
        out_shape=jax.ShapeDtypeStruct((1, _B), jnp.float32),
        scratch_shapes=[pltpu.VMEM((_B, _TC_BLK), jnp.float32)],
    )(boxes_t, boxes_t, boxes_t, boxes_t, boxes_t, gt8)

    mp, dl = pl.pallas_call(
        _tc_finish_body,
        out_shape=[
            jax.ShapeDtypeStruct((1, _B), jnp.float32),
            jax.ShapeDtypeStruct((1, 1), jnp.float32),
        ],
    )(partials, tcp)
    return dl[0, 0], mp.reshape(_B)
